# baseline (device time: 1108481 ns/iter reference)
import jax
import jax.numpy as jnp
from jax import lax
from jax.experimental import pallas as pl
from jax.experimental.pallas import tpu as pltpu

N_DEV = 4


def kernel(x):
    m_per, n = x.shape
    half = m_per // 2
    M = N_DEV * m_per

    def body(
        x_ref, out_ref, stage_r, stage_l,
        copy_sem, stage_sem_r, stage_sem_l, send_r, recv_r, send_l, recv_l,
    ):
        my = lax.axis_index("i")
        left = lax.rem(my + N_DEV - 1, N_DEV)
        right = lax.rem(my + 1, N_DEV)

        barrier_sem = pltpu.get_barrier_semaphore()
        for nbr in (left, right):
            pl.semaphore_signal(
                barrier_sem, inc=1,
                device_id=(nbr,), device_id_type=pl.DeviceIdType.MESH,
            )
        pl.semaphore_wait(barrier_sem, 2)

        local = pltpu.make_async_copy(
            x_ref, out_ref.at[pl.ds(my * m_per, m_per), :], copy_sem
        )
        local.start()

        for h in range(N_DEV - 1):
            orig_r = lax.rem(my - h + N_DEV, N_DEV)
            orig_l = lax.rem(my + h, N_DEV)
            if h == 0:
                src_r = x_ref.at[pl.ds(0, half), :]
                src_l = x_ref.at[pl.ds(half, half), :]
            else:
                src_r = out_ref.at[pl.ds(orig_r * m_per, half), :]
                src_l = out_ref.at[pl.ds(orig_l * m_per + half, half), :]
            cp_r = pltpu.make_async_copy(src_r, stage_r, stage_sem_r)
            cp_l = pltpu.make_async_copy(src_l, stage_l, stage_sem_l)
            cp_r.start()
            cp_l.start()
            cp_r.wait()
            cp_l.wait()
            rdma_r = pltpu.make_async_remote_copy(
                src_ref=stage_r,
                dst_ref=out_ref.at[pl.ds(orig_r * m_per, half), :],
                send_sem=send_r.at[h],
                recv_sem=recv_r.at[h],
                device_id=(right,),
                device_id_type=pl.DeviceIdType.MESH,
            )
            rdma_l = pltpu.make_async_remote_copy(
                src_ref=stage_l,
                dst_ref=out_ref.at[pl.ds(orig_l * m_per + half, half), :],
                send_sem=send_l.at[h],
                recv_sem=recv_l.at[h],
                device_id=(left,),
                device_id_type=pl.DeviceIdType.MESH,
            )
            rdma_r.start()
            rdma_l.start()
            rdma_r.wait()
            rdma_l.wait()

        local.wait()

    return pl.pallas_call(
        body,
        out_shape=jax.ShapeDtypeStruct((M, n), x.dtype),
        in_specs=[pl.BlockSpec(memory_space=pl.ANY)],
        out_specs=pl.BlockSpec(memory_space=pl.ANY),
        scratch_shapes=[
            pltpu.VMEM((half, n), jnp.float32),
            pltpu.VMEM((half, n), jnp.float32),
            pltpu.SemaphoreType.DMA,
            pltpu.SemaphoreType.DMA,
            pltpu.SemaphoreType.DMA,
            pltpu.SemaphoreType.DMA((N_DEV - 1,)),
            pltpu.SemaphoreType.DMA((N_DEV - 1,)),
            pltpu.SemaphoreType.DMA((N_DEV - 1,)),
            pltpu.SemaphoreType.DMA((N_DEV - 1,)),
        ],
        compiler_params=pltpu.CompilerParams(collective_id=0),
    )(x)


# device time: 635053 ns/iter; 1.7455x vs baseline; 1.7455x over previous
import jax
import jax.numpy as jnp
from jax import lax
from jax.experimental import pallas as pl
from jax.experimental.pallas import tpu as pltpu

N_DEV = 4


def kernel(x):
    m_per, n = x.shape
    half = m_per // 2
    M = N_DEV * m_per

    def body(x_ref, out_ref, stage, cp_sem, send_r, recv_r, send_l, recv_l):
        my = lax.axis_index("i")
        left = lax.rem(my + N_DEV - 1, N_DEV)
        right = lax.rem(my + 1, N_DEV)

        barrier_sem = pltpu.get_barrier_semaphore()
        for nbr in (left, right):
            pl.semaphore_signal(
                barrier_sem, inc=1,
                device_id=(nbr,), device_id_type=pl.DeviceIdType.MESH,
            )
        pl.semaphore_wait(barrier_sem, 2)

        for h in range(N_DEV - 1):
            orig_r = lax.rem(my - h + N_DEV, N_DEV)
            orig_l = lax.rem(my + h, N_DEV)
            if h == 0:
                src_r = x_ref.at[pl.ds(0, half), :]
                src_l = x_ref.at[pl.ds(half, half), :]
            else:
                src_r = out_ref.at[pl.ds(orig_r * m_per, half), :]
                src_l = out_ref.at[pl.ds(orig_l * m_per + half, half), :]
            rdma_r = pltpu.make_async_remote_copy(
                src_ref=src_r,
                dst_ref=out_ref.at[pl.ds(orig_r * m_per, half), :],
                send_sem=send_r.at[h],
                recv_sem=recv_r.at[h],
                device_id=(right,),
                device_id_type=pl.DeviceIdType.MESH,
            )
            rdma_l = pltpu.make_async_remote_copy(
                src_ref=src_l,
                dst_ref=out_ref.at[pl.ds(orig_l * m_per + half, half), :],
                send_sem=send_l.at[h],
                recv_sem=recv_l.at[h],
                device_id=(left,),
                device_id_type=pl.DeviceIdType.MESH,
            )
            rdma_r.start()
            rdma_l.start()

            if h == 0:
                cp_in = pltpu.make_async_copy(x_ref, stage, cp_sem)
                cp_in.start()
                cp_in.wait()
                cp_out = pltpu.make_async_copy(
                    stage, out_ref.at[pl.ds(my * m_per, m_per), :], cp_sem
                )
                cp_out.start()
                cp_out.wait()

            rdma_r.wait()
            rdma_l.wait()

    return pl.pallas_call(
        body,
        out_shape=jax.ShapeDtypeStruct((M, n), x.dtype),
        in_specs=[pl.BlockSpec(memory_space=pl.ANY)],
        out_specs=pl.BlockSpec(memory_space=pl.ANY),
        scratch_shapes=[
            pltpu.VMEM((m_per, n), jnp.float32),
            pltpu.SemaphoreType.DMA,
            pltpu.SemaphoreType.DMA((N_DEV - 1,)),
            pltpu.SemaphoreType.DMA((N_DEV - 1,)),
            pltpu.SemaphoreType.DMA((N_DEV - 1,)),
            pltpu.SemaphoreType.DMA((N_DEV - 1,)),
        ],
        compiler_params=pltpu.CompilerParams(
            collective_id=0, vmem_limit_bytes=48 * 1024 * 1024
        ),
    )(x)


# device time: 631037 ns/iter; 1.7566x vs baseline; 1.0064x over previous
import jax
import jax.numpy as jnp
from jax import lax
from jax.experimental import pallas as pl
from jax.experimental.pallas import tpu as pltpu

N_DEV = 4
SUBS = 2
N_SEQ = (N_DEV - 1) * SUBS


def kernel(x):
    m_per, n = x.shape
    half = m_per // 2
    sub = half // SUBS
    M = N_DEV * m_per

    def body(x_ref, out_ref, stage, cp_sem, send_r, recv_r, send_l, recv_l):
        my = lax.axis_index("i")
        left = lax.rem(my + N_DEV - 1, N_DEV)
        right = lax.rem(my + 1, N_DEV)

        barrier_sem = pltpu.get_barrier_semaphore()
        for nbr in (left, right):
            pl.semaphore_signal(
                barrier_sem, inc=1,
                device_id=(nbr,), device_id_type=pl.DeviceIdType.MESH,
            )
        pl.semaphore_wait(barrier_sem, 2)

        rdma_r = []
        rdma_l = []
        for s in range(N_SEQ):
            h, j = divmod(s, SUBS)
            orig_r = lax.rem(my - h + N_DEV, N_DEV)
            orig_l = lax.rem(my + h, N_DEV)
            r_rows = orig_r * m_per + j * sub
            l_rows = orig_l * m_per + half + j * sub
            if h == 0:
                src_r = x_ref.at[pl.ds(j * sub, sub), :]
                src_l = x_ref.at[pl.ds(half + j * sub, sub), :]
            else:
                src_r = out_ref.at[pl.ds(r_rows, sub), :]
                src_l = out_ref.at[pl.ds(l_rows, sub), :]
            rdma_r.append(pltpu.make_async_remote_copy(
                src_ref=src_r,
                dst_ref=out_ref.at[pl.ds(r_rows, sub), :],
                send_sem=send_r.at[s],
                recv_sem=recv_r.at[s],
                device_id=(right,),
                device_id_type=pl.DeviceIdType.MESH,
            ))
            rdma_l.append(pltpu.make_async_remote_copy(
                src_ref=src_l,
                dst_ref=out_ref.at[pl.ds(l_rows, sub), :],
                send_sem=send_l.at[s],
                recv_sem=recv_l.at[s],
                device_id=(left,),
                device_id_type=pl.DeviceIdType.MESH,
            ))

        for s in range(SUBS):
            rdma_r[s].start()
            rdma_l[s].start()

        cp_in = pltpu.make_async_copy(x_ref, stage, cp_sem)
        cp_in.start()
        cp_in.wait()
        cp_out = pltpu.make_async_copy(
            stage, out_ref.at[pl.ds(my * m_per, m_per), :], cp_sem
        )
        cp_out.start()
        cp_out.wait()

        for s in range(SUBS, N_SEQ):
            rdma_r[s - SUBS].wait_recv()
            rdma_l[s - SUBS].wait_recv()
            rdma_r[s].start()
            rdma_l[s].start()

        for s in range(N_SEQ - SUBS, N_SEQ):
            rdma_r[s].wait_recv()
            rdma_l[s].wait_recv()
        for s in range(N_SEQ):
            rdma_r[s].wait_send()
            rdma_l[s].wait_send()

    return pl.pallas_call(
        body,
        out_shape=jax.ShapeDtypeStruct((M, n), x.dtype),
        in_specs=[pl.BlockSpec(memory_space=pl.ANY)],
        out_specs=pl.BlockSpec(memory_space=pl.ANY),
        scratch_shapes=[
            pltpu.VMEM((m_per, n), jnp.float32),
            pltpu.SemaphoreType.DMA,
            pltpu.SemaphoreType.DMA((N_SEQ,)),
            pltpu.SemaphoreType.DMA((N_SEQ,)),
            pltpu.SemaphoreType.DMA((N_SEQ,)),
            pltpu.SemaphoreType.DMA((N_SEQ,)),
        ],
        compiler_params=pltpu.CompilerParams(
            collective_id=0, vmem_limit_bytes=48 * 1024 * 1024
        ),
    )(x)
